# baseline (speedup 1.0000x reference)
import jax
import jax.numpy as jnp
from jax.experimental import pallas as pl
from jax.experimental.pallas import tpu as pltpu


def _round_up(v, m):
    return (v + m - 1) // m * m


_NB = 128


def _lenet_kernel(x_ref,
                  m1, m2,
                  f1k, f1b, f2w, f2b, f3w, f3b,
                  out_ref,
                  x5,
                  x5b):
    f32 = jnp.float32
    nb = _NB

    @pl.when(pl.program_id(0) == 0)
    def _seed():
        x5[...] = jnp.zeros((160, 28 * nb), f32)
        x5[28:29, :] = jnp.ones((1, 28 * nb), f32)
        x5b[...] = jnp.zeros((448, 10 * nb), f32)
        x5b[84:85, :] = jnp.ones((1, 10 * nb), f32)

    xv = x_ref[0]
    for j in range(5):
        lo = max(0, 2 - j)
        hi = min(28, 30 - j)
        x5[j * 32: j * 32 + 28, lo * nb: hi * nb] = (
            xv[:, (lo + j - 2) * nb: (hi + j - 2) * nb])

    a1 = jnp.dot(m1[...], x5[...], preferred_element_type=f32)

    for t in range(14):
        c0, c1 = 2 * t * nb, (2 * t + 1) * nb
        chunk = jnp.maximum(
            jnp.maximum(jnp.maximum(a1[:84, c0:c1], a1[:84, c1:c1 + nb]),
                        jnp.maximum(a1[88:172, c0:c1],
                                    a1[88:172, c1:c1 + nb])), 0.0)
        for j in range(5):
            tp = t - j
            if 0 <= tp <= 9:
                x5b[j * 88: j * 88 + 84, tp * nb:(tp + 1) * nb] = chunk

    a2 = jnp.dot(m2[...], x5b[...], preferred_element_type=f32)

    mz = jnp.concatenate(
        [jnp.maximum(
            jnp.maximum(
                jnp.maximum(a2[:80, 2 * t * nb:(2 * t + 1) * nb],
                            a2[:80, (2 * t + 1) * nb:(2 * t + 2) * nb]),
                jnp.maximum(a2[88:168, 2 * t * nb:(2 * t + 1) * nb],
                            a2[88:168, (2 * t + 1) * nb:(2 * t + 2) * nb])),
            0.0)
         for t in range(5)], axis=1)

    h1 = jnp.dot(f1k[0], mz[:, :nb], preferred_element_type=f32)
    for k in range(1, 5):
        h1 = h1 + jnp.dot(f1k[k], mz[:, k * nb:(k + 1) * nb],
                          preferred_element_type=f32)
    h1 = jnp.maximum(h1 + f1b[...], 0.0)
    h2 = jnp.maximum(jnp.dot(f2w[...], h1, preferred_element_type=f32)
                     + f2b[...], 0.0)
    logits = (jnp.dot(f3w[...], h2, preferred_element_type=f32)
              + f3b[...])
    out_ref[...] = jnp.transpose(logits)


@jax.jit
def _lenet_forward(x, conv1_w, conv1_b, conv2_w, conv2_b,
                   lin1_w, lin1_b, lin2_w, lin2_b, lin3_w, lin3_b):
    f32 = jnp.float32
    B = x.shape[0]
    nb = _NB
    bp = _round_up(B, nb)
    nblk = bp // nb

    xr = x.astype(f32).reshape(B, 784)
    if bp != B:
        xr = jnp.pad(xr, ((0, bp - B), (0, 0)))
    xin = (xr.reshape(nblk, nb, 784).transpose(0, 2, 1)
           .reshape(nblk, 28, 28 * nb))

    def rowmap(chans, rows):
        o = jnp.arange(chans * rows) // rows
        y = jnp.arange(chans * rows) % rows
        return o * (rows // 2) + y // 2 + 88 * (y % 2)

    i_ = jnp.arange(5)[:, None, None]
    y_ = jnp.arange(28)[None, :, None]
    k_ = jnp.arange(28)[None, None, :]
    e1 = ((k_ == y_ + i_ - 2) & (y_ + i_ - 2 >= 0)).astype(f32)
    w1 = conv1_w.reshape(6, 5, 5).astype(f32)
    m1full = jnp.einsum('oij,iyk->joyk', w1, e1).reshape(5, 168, 28)
    m1p = jnp.zeros((5, 176, 28), f32).at[:, rowmap(6, 28), :].set(m1full)
    cols = (jnp.arange(5)[:, None] * 32 + jnp.arange(28)[None, :]).ravel()
    b1vec = (jnp.zeros((176,), f32)
             .at[rowmap(6, 28)].set(jnp.repeat(conv1_b.astype(f32), 28)))
    m1 = (jnp.zeros((176, 160), f32)
          .at[:, cols].set(m1p.transpose(1, 0, 2).reshape(176, 140))
          .at[:, 28].set(b1vec))

    y2_ = jnp.arange(10)[None, :, None]
    k2_ = jnp.arange(14)[None, None, :]
    e2 = (k2_ == y2_ + i_).astype(f32)
    w2 = conv2_w.astype(f32)
    m2full = jnp.einsum('ocij,iyk->joyck', w2, e2).reshape(5, 160, 84)
    m2p = jnp.zeros((5, 168, 84), f32).at[:, rowmap(16, 10), :].set(m2full)
    cols2 = (jnp.arange(5)[:, None] * 88 + jnp.arange(84)[None, :]).ravel()
    b2vec = (jnp.zeros((168,), f32)
             .at[rowmap(16, 10)].set(jnp.repeat(conv2_b.astype(f32), 10)))
    m2 = (jnp.zeros((168, 448), f32)
          .at[:, cols2].set(m2p.transpose(1, 0, 2).reshape(168, 420))
          .at[:, 84].set(b2vec))

    kk, oo, yy = jnp.meshgrid(jnp.arange(5), jnp.arange(16), jnp.arange(5),
                              indexing='ij')
    kk, oo, yy = kk.ravel(), oo.ravel(), yy.ravel()
    f1w = lin1_w.astype(f32)
    f1k = (jnp.zeros((5, 100, 80), f32)
           .at[kk, :, oo * 5 + yy].set(f1w[:, oo * 25 + yy * 5 + kk].T))

    f1b = lin1_b.astype(f32).reshape(100, 1)
    f2w = lin2_w.astype(f32)
    f2b = lin2_b.astype(f32).reshape(50, 1)
    f3w = lin3_w.astype(f32)
    f3b = lin3_b.astype(f32).reshape(10, 1)

    def resident(shape):
        n = len(shape)
        return pl.BlockSpec(shape, lambda b: (0,) * n)

    out = pl.pallas_call(
        _lenet_kernel,
        out_shape=jax.ShapeDtypeStruct((bp, 10), f32),
        grid=(nblk,),
        in_specs=[
            pl.BlockSpec((1, 28, 28 * nb), lambda b: (b, 0, 0)),
            resident((176, 160)),
            resident((168, 448)),
            resident((5, 100, 80)), resident((100, 1)),
            resident((50, 100)), resident((50, 1)),
            resident((10, 50)), resident((10, 1)),
        ],
        out_specs=pl.BlockSpec((nb, 10), lambda b: (b, 0)),
        scratch_shapes=[pltpu.VMEM((160, 28 * nb), f32),
                        pltpu.VMEM((448, 10 * nb), f32)],
        compiler_params=pltpu.CompilerParams(
            dimension_semantics=("parallel",),
            vmem_limit_bytes=64 * 1024 * 1024,
        ),
    )(xin, m1, m2, f1k, f1b, f2w, f2b, f3w, f3b)

    return out[:B]


def kernel(x, conv1_w, conv1_b, conv2_w, conv2_b,
           lin1_w, lin1_b, lin2_w, lin2_b, lin3_w, lin3_b):
    return _lenet_forward(x, conv1_w, conv1_b, conv2_w, conv2_b,
                          lin1_w, lin1_b, lin2_w, lin2_b, lin3_w, lin3_b)


# speedup vs baseline: 1.0682x; 1.0682x over previous
import jax
import jax.numpy as jnp
from jax.experimental import pallas as pl
from jax.experimental.pallas import tpu as pltpu


def _round_up(v, m):
    return (v + m - 1) // m * m


_NB = 128


def _lenet_kernel(x_ref,
                  m1, m2,
                  f1k, f1b, f2w, f2b, f3w, f3b,
                  out_ref,
                  x5,
                  x5b):
    f32 = jnp.float32
    nb = _NB

    @pl.when(pl.program_id(0) == 0)
    def _seed():
        x5[...] = jnp.zeros((160, 28 * nb), f32)
        x5[28:29, :] = jnp.ones((1, 28 * nb), f32)
        x5b[...] = jnp.zeros((448, 10 * nb), f32)
        x5b[84:85, :] = jnp.ones((1, 10 * nb), f32)

    xv = x_ref[0]
    for j in range(5):
        lo = max(0, 2 - j)
        hi = min(28, 30 - j)
        x5[j * 32: j * 32 + 28, lo * nb: hi * nb] = (
            xv[:, (lo + j - 2) * nb: (hi + j - 2) * nb])

    a1 = jnp.dot(m1[...], x5[...], preferred_element_type=f32)

    for t in range(14):
        c0, c1 = 2 * t * nb, (2 * t + 1) * nb
        chunk = jnp.maximum(
            jnp.maximum(jnp.maximum(a1[:84, c0:c1], a1[:84, c1:c1 + nb]),
                        jnp.maximum(a1[88:172, c0:c1],
                                    a1[88:172, c1:c1 + nb])), 0.0)
        for j in range(5):
            tp = t - j
            if 0 <= tp <= 9:
                x5b[j * 88: j * 88 + 84, tp * nb:(tp + 1) * nb] = chunk

    a2 = jnp.dot(m2[...], x5b[...], preferred_element_type=f32)

    mz = jnp.concatenate(
        [jnp.maximum(
            jnp.maximum(
                jnp.maximum(a2[:80, 2 * t * nb:(2 * t + 1) * nb],
                            a2[:80, (2 * t + 1) * nb:(2 * t + 2) * nb]),
                jnp.maximum(a2[88:168, 2 * t * nb:(2 * t + 1) * nb],
                            a2[88:168, (2 * t + 1) * nb:(2 * t + 2) * nb])),
            0.0)
         for t in range(5)], axis=1)

    h1 = jnp.dot(f1k[0], mz[:, :nb], preferred_element_type=f32)
    for k in range(1, 5):
        h1 = h1 + jnp.dot(f1k[k], mz[:, k * nb:(k + 1) * nb],
                          preferred_element_type=f32)
    h1 = jnp.maximum(h1 + f1b[...], 0.0)
    h2 = jnp.maximum(jnp.dot(f2w[...], h1, preferred_element_type=f32)
                     + f2b[...], 0.0)
    out_ref[...] = (jnp.dot(f3w[...], h2, preferred_element_type=f32)
                    + f3b[...])


@jax.jit
def _lenet_forward(x, conv1_w, conv1_b, conv2_w, conv2_b,
                   lin1_w, lin1_b, lin2_w, lin2_b, lin3_w, lin3_b):
    f32 = jnp.float32
    B = x.shape[0]
    nb = _NB
    bp = _round_up(B, nb)
    nblk = bp // nb

    xr = x.astype(f32).reshape(B, 784)
    if bp != B:
        xr = jnp.pad(xr, ((0, bp - B), (0, 0)))
    xin = (xr.reshape(nblk, nb, 784).transpose(0, 2, 1)
           .reshape(nblk, 28, 28 * nb))

    def rowmap(chans, rows):
        o = jnp.arange(chans * rows) // rows
        y = jnp.arange(chans * rows) % rows
        return o * (rows // 2) + y // 2 + 88 * (y % 2)

    i_ = jnp.arange(5)[:, None, None]
    y_ = jnp.arange(28)[None, :, None]
    k_ = jnp.arange(28)[None, None, :]
    e1 = ((k_ == y_ + i_ - 2) & (y_ + i_ - 2 >= 0)).astype(f32)
    w1 = conv1_w.reshape(6, 5, 5).astype(f32)
    m1full = jnp.einsum('oij,iyk->joyk', w1, e1).reshape(5, 168, 28)
    m1p = jnp.zeros((5, 176, 28), f32).at[:, rowmap(6, 28), :].set(m1full)
    cols = (jnp.arange(5)[:, None] * 32 + jnp.arange(28)[None, :]).ravel()
    b1vec = (jnp.zeros((176,), f32)
             .at[rowmap(6, 28)].set(jnp.repeat(conv1_b.astype(f32), 28)))
    m1 = (jnp.zeros((176, 160), f32)
          .at[:, cols].set(m1p.transpose(1, 0, 2).reshape(176, 140))
          .at[:, 28].set(b1vec))

    y2_ = jnp.arange(10)[None, :, None]
    k2_ = jnp.arange(14)[None, None, :]
    e2 = (k2_ == y2_ + i_).astype(f32)
    w2 = conv2_w.astype(f32)
    m2full = jnp.einsum('ocij,iyk->joyck', w2, e2).reshape(5, 160, 84)
    m2p = jnp.zeros((5, 168, 84), f32).at[:, rowmap(16, 10), :].set(m2full)
    cols2 = (jnp.arange(5)[:, None] * 88 + jnp.arange(84)[None, :]).ravel()
    b2vec = (jnp.zeros((168,), f32)
             .at[rowmap(16, 10)].set(jnp.repeat(conv2_b.astype(f32), 10)))
    m2 = (jnp.zeros((168, 448), f32)
          .at[:, cols2].set(m2p.transpose(1, 0, 2).reshape(168, 420))
          .at[:, 84].set(b2vec))

    kk, oo, yy = jnp.meshgrid(jnp.arange(5), jnp.arange(16), jnp.arange(5),
                              indexing='ij')
    kk, oo, yy = kk.ravel(), oo.ravel(), yy.ravel()
    f1w = lin1_w.astype(f32)
    f1k = (jnp.zeros((5, 100, 80), f32)
           .at[kk, :, oo * 5 + yy].set(f1w[:, oo * 25 + yy * 5 + kk].T))

    f1b = lin1_b.astype(f32).reshape(100, 1)
    f2w = lin2_w.astype(f32)
    f2b = lin2_b.astype(f32).reshape(50, 1)
    f3w = lin3_w.astype(f32)
    f3b = lin3_b.astype(f32).reshape(10, 1)

    def resident(shape):
        n = len(shape)
        return pl.BlockSpec(shape, lambda b: (0,) * n)

    out = pl.pallas_call(
        _lenet_kernel,
        out_shape=jax.ShapeDtypeStruct((10, bp), f32),
        grid=(nblk,),
        in_specs=[
            pl.BlockSpec((1, 28, 28 * nb), lambda b: (b, 0, 0)),
            resident((176, 160)),
            resident((168, 448)),
            resident((5, 100, 80)), resident((100, 1)),
            resident((50, 100)), resident((50, 1)),
            resident((10, 50)), resident((10, 1)),
        ],
        out_specs=pl.BlockSpec((10, nb), lambda b: (0, b)),
        scratch_shapes=[pltpu.VMEM((160, 28 * nb), f32),
                        pltpu.VMEM((448, 10 * nb), f32)],
        compiler_params=pltpu.CompilerParams(
            dimension_semantics=("parallel",),
            vmem_limit_bytes=64 * 1024 * 1024,
        ),
    )(xin, m1, m2, f1k, f1b, f2w, f2b, f3w, f3b)

    return out[:, :B].T


def kernel(x, conv1_w, conv1_b, conv2_w, conv2_b,
           lin1_w, lin1_b, lin2_w, lin2_b, lin3_w, lin3_b):
    return _lenet_forward(x, conv1_w, conv1_b, conv2_w, conv2_b,
                          lin1_w, lin1_b, lin2_w, lin2_b, lin3_w, lin3_b)
